# baseline (speedup 1.0000x reference)
import functools

import jax
import jax.numpy as jnp
from jax import lax
from jax.experimental import pallas as pl
from jax.experimental.pallas import tpu as pltpu
from jax.experimental.pallas import tpu_sc as plsc

N = 10000
E = 320000
F = 128
C = 64

NPAD = 10240
ROWS_PER_TILE = NPAD // 16
CHUNK = 128
NCHK = ((E + CHUNK - 1) // CHUNK + 255) // 256 * 256
EPAD = NCHK * CHUNK
CHUNKS_PER_TILE = NCHK // 32
CHUNKS_PER_SUBCORE = NCHK // 16
FH = F // 2
IDXB = 16
DEGW = 16


def _sc_agg_body(h_hbm, src_hbm, dst_hbm, agg_out, src_v, dst_v,
                 rows_v0, rows_v1, rows_v2, rows_v3, table_sp, acc_sp,
                 sem_g0, sem_g1, sem_g2, sem_g3,
                 sem_s0, sem_s1, sem_s2, sem_s3):
    cid = lax.axis_index("c")
    tid = lax.axis_index("s")

    def fill(i, _):
        for g in range(FH // 16):
            rows_v0[i, pl.ds(g * 16, 16)] = jnp.zeros((16,), jnp.float32)
        return 0
    lax.fori_loop(0, CHUNK, fill, 0)

    my0 = tid * ROWS_PER_TILE
    for k in range(ROWS_PER_TILE // CHUNK):
        pltpu.sync_copy(rows_v0, acc_sp.at[pl.ds(my0 + k * CHUNK, CHUNK)])

    @pl.when(cid == 0)
    def _():
        pltpu.sync_copy(h_hbm.at[pl.ds(my0, ROWS_PER_TILE), pl.ds(0, FH)],
                        table_sp.at[pl.ds(my0, ROWS_PER_TILE)])

    @pl.when(cid == 1)
    def _():
        pltpu.sync_copy(h_hbm.at[pl.ds(my0, ROWS_PER_TILE), pl.ds(FH, FH)],
                        table_sp.at[pl.ds(my0, ROWS_PER_TILE)])
    plsc.subcore_barrier()

    rows = (rows_v0, rows_v1, rows_v2, rows_v3)
    gsem = (sem_g0, sem_g1, sem_g2, sem_g3)
    ssem = (sem_s0, sem_s1, sem_s2, sem_s3)

    def outer(g, _):
        base = tid * CHUNKS_PER_SUBCORE + g * IDXB
        pltpu.sync_copy(src_hbm.at[pl.ds(base, IDXB)], src_v)
        pltpu.sync_copy(dst_hbm.at[pl.ds(base, IDXB)], dst_v)
        for k in range(4):
            pltpu.async_copy(table_sp.at[src_v.at[k]], rows[k], gsem[k])

        def inner(jj, _):
            j = jj * 4
            for k in range(4):
                pltpu.make_async_copy(table_sp.at[src_v.at[0]],
                                      rows[k], gsem[k]).wait()
                pltpu.async_copy(rows[k], acc_sp.at[dst_v.at[j + k]],
                                 ssem[k], add=True)
            for k in range(4):
                pltpu.make_async_copy(rows[k], acc_sp.at[dst_v.at[0]],
                                      ssem[k]).wait()

                @pl.when(j + 4 + k < IDXB)
                def _():
                    pltpu.async_copy(table_sp.at[src_v.at[j + 4 + k]],
                                     rows[k], gsem[k])
            return 0
        lax.fori_loop(0, IDXB // 4, inner, 0)
        return 0
    lax.fori_loop(0, CHUNKS_PER_SUBCORE // IDXB, outer, 0)

    plsc.subcore_barrier()

    out0 = cid * NPAD + my0
    pltpu.sync_copy(acc_sp.at[pl.ds(my0, ROWS_PER_TILE)],
                    agg_out.at[pl.ds(out0, ROWS_PER_TILE)])


def _sc_deg_body(dst_hbm, deg_out, dst_v, ones_v, zbuf_v, deg_sp):
    cid = lax.axis_index("c")
    tid = lax.axis_index("s")
    wid = cid * 16 + tid

    def fill(i, _):
        ones_v[i, :] = jnp.ones((DEGW,), jnp.float32)
        zbuf_v[i, :] = jnp.zeros((DEGW,), jnp.float32)
        return 0
    lax.fori_loop(0, CHUNK, fill, 0)

    my0 = tid * ROWS_PER_TILE
    for k in range(ROWS_PER_TILE // CHUNK):
        pltpu.sync_copy(zbuf_v, deg_sp.at[pl.ds(my0 + k * CHUNK, CHUNK)])
    plsc.subcore_barrier()

    pltpu.sync_copy(dst_hbm.at[pl.ds(wid * CHUNKS_PER_TILE, CHUNKS_PER_TILE)], dst_v)

    def edge_body(j, _):
        pltpu.sync_copy(ones_v, deg_sp.at[dst_v.at[j]], add=True)
        return 0
    lax.fori_loop(0, CHUNKS_PER_TILE, edge_body, 0)

    plsc.subcore_barrier()

    out0 = cid * NPAD + my0
    pltpu.sync_copy(deg_sp.at[pl.ds(my0, ROWS_PER_TILE)],
                    deg_out.at[pl.ds(out0, ROWS_PER_TILE)])


_SC_PARAMS = pltpu.CompilerParams(use_tc_tiling_on_sc=False)
_SC_MESH = dict(core_axis_name="c", subcore_axis_name="s")

_sc_agg = pl.kernel(
    _sc_agg_body,
    out_type=jax.ShapeDtypeStruct((2 * NPAD, FH), jnp.float32),
    mesh=plsc.VectorSubcoreMesh(**_SC_MESH),
    scratch_types=[
        pltpu.VMEM((IDXB, CHUNK), jnp.int32),
        pltpu.VMEM((IDXB, CHUNK), jnp.int32),
        pltpu.VMEM((CHUNK, FH), jnp.float32),
        pltpu.VMEM((CHUNK, FH), jnp.float32),
        pltpu.VMEM((CHUNK, FH), jnp.float32),
        pltpu.VMEM((CHUNK, FH), jnp.float32),
        pltpu.VMEM_SHARED((NPAD, FH), jnp.float32),
        pltpu.VMEM_SHARED((NPAD, FH), jnp.float32),
    ] + [pltpu.SemaphoreType.DMA] * 8,
    compiler_params=_SC_PARAMS,
)

_sc_deg = pl.kernel(
    _sc_deg_body,
    out_type=jax.ShapeDtypeStruct((2 * NPAD, DEGW), jnp.float32),
    mesh=plsc.VectorSubcoreMesh(**_SC_MESH),
    scratch_types=[
        pltpu.VMEM((CHUNKS_PER_TILE, CHUNK), jnp.int32),
        pltpu.VMEM((CHUNK, DEGW), jnp.float32),
        pltpu.VMEM((CHUNK, DEGW), jnp.float32),
        pltpu.VMEM_SHARED((NPAD, DEGW), jnp.float32),
    ],
    compiler_params=_SC_PARAMS,
)

_BLK = 1280
_NB = NPAD // _BLK


def _tc_layer1_body(x, a0, a1, d0, d1, wl, wr0, wr1, b, o):
    inv = 1.0 / jnp.clip(d0[..., :1] + d1[..., :1], 1.0, None)
    h = (jnp.dot(x[...], wl[...], preferred_element_type=jnp.float32)
         + jnp.dot(a0[...] * inv, wr0[...], preferred_element_type=jnp.float32)
         + jnp.dot(a1[...] * inv, wr1[...], preferred_element_type=jnp.float32)
         + b[...])
    o[...] = jnp.maximum(h, 0.0)


def _tc_layer2_body(x, a0, a1, d0, d1, wl, wr0, wr1, b, wm, bm, o):
    inv = 1.0 / jnp.clip(d0[..., :1] + d1[..., :1], 1.0, None)
    h = (jnp.dot(x[...], wl[...], preferred_element_type=jnp.float32)
         + jnp.dot(a0[...] * inv, wr0[...], preferred_element_type=jnp.float32)
         + jnp.dot(a1[...] * inv, wr1[...], preferred_element_type=jnp.float32)
         + b[...])
    h = jnp.maximum(h, 0.0)
    logits = jnp.dot(h, wm[...], preferred_element_type=jnp.float32) + bm[...]
    m = jnp.max(logits, axis=1, keepdims=True)
    s = logits - m
    lse = jnp.log(jnp.sum(jnp.exp(s), axis=1, keepdims=True))
    o[...] = s - lse


def _row_spec(w):
    return pl.BlockSpec((_BLK, w), lambda i: (i, 0))


def _hi_spec(w):
    return pl.BlockSpec((_BLK, w), lambda i: (i + _NB, 0))


def _full_spec(h, w):
    return pl.BlockSpec((h, w), lambda i: (0, 0))


def _tc_layer1(x, agg, deg, wl, wr0, wr1, b):
    return pl.pallas_call(
        _tc_layer1_body,
        grid=(_NB,),
        in_specs=[_row_spec(F), _row_spec(FH), _hi_spec(FH),
                  _row_spec(DEGW), _hi_spec(DEGW),
                  _full_spec(F, F), _full_spec(FH, F), _full_spec(FH, F),
                  _full_spec(1, F)],
        out_specs=_row_spec(F),
        out_shape=jax.ShapeDtypeStruct((NPAD, F), jnp.float32),
    )(x, agg, agg, deg, deg, wl, wr0, wr1, b)


def _tc_layer2(x, agg, deg, wl, wr0, wr1, b, wm, bm):
    return pl.pallas_call(
        _tc_layer2_body,
        grid=(_NB,),
        in_specs=[_row_spec(F), _row_spec(FH), _hi_spec(FH),
                  _row_spec(DEGW), _hi_spec(DEGW),
                  _full_spec(F, F), _full_spec(FH, F), _full_spec(FH, F),
                  _full_spec(1, F), _full_spec(F, C), _full_spec(1, C)],
        out_specs=_row_spec(C),
        out_shape=jax.ShapeDtypeStruct((NPAD, C), jnp.float32),
    )(x, agg, agg, deg, deg, wl, wr0, wr1, b, wm, bm)


def kernel(x, edge_index, W1l, W1r, b1, W2l, W2r, b2, Wm, bm):
    src = edge_index[0]
    dst = edge_index[1]
    pad = EPAD - E
    src_p = jnp.concatenate([src, jnp.zeros((pad,), jnp.int32)]).reshape(NCHK, CHUNK)
    dst_p = jnp.concatenate([dst, jnp.full((pad,), N, jnp.int32)]).reshape(NCHK, CHUNK)

    xp = jnp.concatenate([x, jnp.zeros((NPAD - N, F), jnp.float32)])

    agg1 = _sc_agg(xp, src_p, dst_p)
    deg = _sc_deg(dst_p)

    h1 = _tc_layer1(xp, agg1, deg, W1l, W1r[:FH], W1r[FH:], b1.reshape(1, F))

    agg2 = _sc_agg(h1, src_p, dst_p)

    out = _tc_layer2(h1, agg2, deg, W2l, W2r[:FH], W2r[FH:],
                     b2.reshape(1, F), Wm, bm.reshape(1, C))
    return out[:N]


# speedup vs baseline: 1.0128x; 1.0128x over previous
import functools

import jax
import jax.numpy as jnp
from jax import lax
from jax.experimental import pallas as pl
from jax.experimental.pallas import tpu as pltpu
from jax.experimental.pallas import tpu_sc as plsc

N = 10000
E = 320000
F = 128
C = 64

NPAD = 10240
ROWS_PER_TILE = NPAD // 16
CHUNK = 128
NCHK = ((E + CHUNK - 1) // CHUNK + 255) // 256 * 256
EPAD = NCHK * CHUNK
CHUNKS_PER_TILE = NCHK // 32
CHUNKS_PER_SUBCORE = NCHK // 16
FH = F // 2
IDXB = 16
DEGW = 16


def _sc_agg_body(compute_deg, h_hbm, src_hbm, dst_hbm, *refs):
    if compute_deg:
        (agg_out, deg_out, src_v, dst_v, rows_v0, rows_v1, ones_v,
         table_sp, acc_sp, deg_sp, sem0, sem1) = refs
    else:
        (agg_out, src_v, dst_v, rows_v0, rows_v1,
         table_sp, acc_sp, sem0, sem1) = refs
    cid = lax.axis_index("c")
    tid = lax.axis_index("s")

    def fill(i, _):
        for g in range(FH // 16):
            rows_v0[i, pl.ds(g * 16, 16)] = jnp.zeros((16,), jnp.float32)
        if compute_deg:
            ones_v[i, :] = jnp.ones((DEGW,), jnp.float32)
        return 0
    lax.fori_loop(0, CHUNK, fill, 0)

    my0 = tid * ROWS_PER_TILE
    for k in range(ROWS_PER_TILE // CHUNK):
        pltpu.sync_copy(rows_v0, acc_sp.at[pl.ds(my0 + k * CHUNK, CHUNK)])
        if compute_deg:
            pltpu.sync_copy(rows_v0.at[pl.ds(0, CHUNK), pl.ds(0, DEGW)],
                            deg_sp.at[pl.ds(my0 + k * CHUNK, CHUNK)])

    @pl.when(cid == 0)
    def _():
        pltpu.sync_copy(h_hbm.at[pl.ds(my0, ROWS_PER_TILE), pl.ds(0, FH)],
                        table_sp.at[pl.ds(my0, ROWS_PER_TILE)])

    @pl.when(cid == 1)
    def _():
        pltpu.sync_copy(h_hbm.at[pl.ds(my0, ROWS_PER_TILE), pl.ds(FH, FH)],
                        table_sp.at[pl.ds(my0, ROWS_PER_TILE)])
    plsc.subcore_barrier()

    def outer(g, _):
        base = tid * CHUNKS_PER_SUBCORE + g * IDXB
        pltpu.sync_copy(src_hbm.at[pl.ds(base, IDXB)], src_v)
        pltpu.sync_copy(dst_hbm.at[pl.ds(base, IDXB)], dst_v)
        pltpu.async_copy(table_sp.at[src_v.at[0]], rows_v0, sem0)

        def inner(jj, _):
            j = jj * 2
            pltpu.make_async_copy(table_sp.at[src_v.at[0]], rows_v0, sem0).wait()
            pltpu.async_copy(table_sp.at[src_v.at[j + 1]], rows_v1, sem1)
            pltpu.sync_copy(rows_v0, acc_sp.at[dst_v.at[j]], add=True)
            if compute_deg:
                @pl.when((g + jj) % 2 == cid)
                def _():
                    pltpu.sync_copy(ones_v, deg_sp.at[dst_v.at[j]], add=True)

            pltpu.make_async_copy(table_sp.at[src_v.at[0]], rows_v1, sem1).wait()

            @pl.when(jj + 1 < IDXB // 2)
            def _():
                pltpu.async_copy(table_sp.at[src_v.at[j + 2]], rows_v0, sem0)
            pltpu.sync_copy(rows_v1, acc_sp.at[dst_v.at[j + 1]], add=True)
            if compute_deg:
                @pl.when((g + jj) % 2 != cid)
                def _():
                    pltpu.sync_copy(ones_v, deg_sp.at[dst_v.at[j + 1]], add=True)
            return 0
        lax.fori_loop(0, IDXB // 2, inner, 0)
        return 0
    lax.fori_loop(0, CHUNKS_PER_SUBCORE // IDXB, outer, 0)

    plsc.subcore_barrier()

    out0 = cid * NPAD + my0
    pltpu.sync_copy(acc_sp.at[pl.ds(my0, ROWS_PER_TILE)],
                    agg_out.at[pl.ds(out0, ROWS_PER_TILE)])
    if compute_deg:
        pltpu.sync_copy(deg_sp.at[pl.ds(my0, ROWS_PER_TILE)],
                        deg_out.at[pl.ds(out0, ROWS_PER_TILE)])


_SC_PARAMS = pltpu.CompilerParams(use_tc_tiling_on_sc=False)
_SC_MESH = dict(core_axis_name="c", subcore_axis_name="s")

def _make_sc_agg(compute_deg):
    out_type = [jax.ShapeDtypeStruct((2 * NPAD, FH), jnp.float32)]
    scratch = [
        pltpu.VMEM((IDXB, CHUNK), jnp.int32),
        pltpu.VMEM((IDXB, CHUNK), jnp.int32),
        pltpu.VMEM((CHUNK, FH), jnp.float32),
        pltpu.VMEM((CHUNK, FH), jnp.float32),
    ]
    if compute_deg:
        out_type.append(jax.ShapeDtypeStruct((2 * NPAD, DEGW), jnp.float32))
        scratch.append(pltpu.VMEM((CHUNK, DEGW), jnp.float32))
    scratch.append(pltpu.VMEM_SHARED((NPAD, FH), jnp.float32))
    scratch.append(pltpu.VMEM_SHARED((NPAD, FH), jnp.float32))
    if compute_deg:
        scratch.append(pltpu.VMEM_SHARED((NPAD, DEGW), jnp.float32))
    scratch += [pltpu.SemaphoreType.DMA, pltpu.SemaphoreType.DMA]
    return pl.kernel(
        functools.partial(_sc_agg_body, compute_deg),
        out_type=out_type if compute_deg else out_type[0],
        mesh=plsc.VectorSubcoreMesh(**_SC_MESH),
        scratch_types=scratch,
        compiler_params=_SC_PARAMS,
    )


_sc_agg_deg = _make_sc_agg(True)
_sc_agg = _make_sc_agg(False)

_BLK = 1280
_NB = NPAD // _BLK


def _tc_layer1_body(x, a0, a1, d0, d1, wl, wr0, wr1, b, o):
    inv = 1.0 / jnp.clip(d0[..., :1] + d1[..., :1], 1.0, None)
    h = (jnp.dot(x[...], wl[...], preferred_element_type=jnp.float32)
         + jnp.dot(a0[...] * inv, wr0[...], preferred_element_type=jnp.float32)
         + jnp.dot(a1[...] * inv, wr1[...], preferred_element_type=jnp.float32)
         + b[...])
    o[...] = jnp.maximum(h, 0.0)


def _tc_layer2_body(x, a0, a1, d0, d1, wl, wr0, wr1, b, wm, bm, o):
    inv = 1.0 / jnp.clip(d0[..., :1] + d1[..., :1], 1.0, None)
    h = (jnp.dot(x[...], wl[...], preferred_element_type=jnp.float32)
         + jnp.dot(a0[...] * inv, wr0[...], preferred_element_type=jnp.float32)
         + jnp.dot(a1[...] * inv, wr1[...], preferred_element_type=jnp.float32)
         + b[...])
    h = jnp.maximum(h, 0.0)
    logits = jnp.dot(h, wm[...], preferred_element_type=jnp.float32) + bm[...]
    m = jnp.max(logits, axis=1, keepdims=True)
    s = logits - m
    lse = jnp.log(jnp.sum(jnp.exp(s), axis=1, keepdims=True))
    o[...] = s - lse


def _row_spec(w):
    return pl.BlockSpec((_BLK, w), lambda i: (i, 0))


def _hi_spec(w):
    return pl.BlockSpec((_BLK, w), lambda i: (i + _NB, 0))


def _full_spec(h, w):
    return pl.BlockSpec((h, w), lambda i: (0, 0))


def _tc_layer1(x, agg, deg, wl, wr0, wr1, b):
    return pl.pallas_call(
        _tc_layer1_body,
        grid=(_NB,),
        in_specs=[_row_spec(F), _row_spec(FH), _hi_spec(FH),
                  _row_spec(DEGW), _hi_spec(DEGW),
                  _full_spec(F, F), _full_spec(FH, F), _full_spec(FH, F),
                  _full_spec(1, F)],
        out_specs=_row_spec(F),
        out_shape=jax.ShapeDtypeStruct((NPAD, F), jnp.float32),
    )(x, agg, agg, deg, deg, wl, wr0, wr1, b)


def _tc_layer2(x, agg, deg, wl, wr0, wr1, b, wm, bm):
    return pl.pallas_call(
        _tc_layer2_body,
        grid=(_NB,),
        in_specs=[_row_spec(F), _row_spec(FH), _hi_spec(FH),
                  _row_spec(DEGW), _hi_spec(DEGW),
                  _full_spec(F, F), _full_spec(FH, F), _full_spec(FH, F),
                  _full_spec(1, F), _full_spec(F, C), _full_spec(1, C)],
        out_specs=_row_spec(C),
        out_shape=jax.ShapeDtypeStruct((NPAD, C), jnp.float32),
    )(x, agg, agg, deg, deg, wl, wr0, wr1, b, wm, bm)


def kernel(x, edge_index, W1l, W1r, b1, W2l, W2r, b2, Wm, bm):
    src = edge_index[0]
    dst = edge_index[1]
    pad = EPAD - E
    src_p = jnp.concatenate([src, jnp.zeros((pad,), jnp.int32)]).reshape(NCHK, CHUNK)
    dst_p = jnp.concatenate([dst, jnp.full((pad,), N, jnp.int32)]).reshape(NCHK, CHUNK)

    xp = jnp.concatenate([x, jnp.zeros((NPAD - N, F), jnp.float32)])

    agg1, deg = _sc_agg_deg(xp, src_p, dst_p)

    h1 = _tc_layer1(xp, agg1, deg, W1l, W1r[:FH], W1r[FH:], b1.reshape(1, F))

    agg2 = _sc_agg(h1, src_p, dst_p)

    out = _tc_layer2(h1, agg2, deg, W2l, W2r[:FH], W2r[FH:],
                     b2.reshape(1, F), Wm, bm.reshape(1, C))
    return out[:N]


# speedup vs baseline: 1.0401x; 1.0270x over previous
import functools

import jax
import jax.numpy as jnp
from jax import lax
from jax.experimental import pallas as pl
from jax.experimental.pallas import tpu as pltpu
from jax.experimental.pallas import tpu_sc as plsc

N = 10000
E = 320000
F = 128
C = 64

NPAD = 10240
ROWS_PER_TILE = NPAD // 16
CHUNK = 128
NCHK = ((E + CHUNK - 1) // CHUNK + 255) // 256 * 256
EPAD = NCHK * CHUNK
CHUNKS_PER_TILE = NCHK // 32
CHUNKS_PER_SUBCORE = NCHK // 16
FH = F // 2
IDXB = 16
DEGW = 16


def _sc_agg_body(compute_deg, h_hbm, src_hbm, dst_hbm, *refs):
    if compute_deg:
        (agg_out, deg_out, src_v, dst_v, rows_v0, rows_v1, ones_v,
         table_sp, acc_sp, deg_sp, sem0, sem1) = refs
    else:
        (agg_out, src_v, dst_v, rows_v0, rows_v1,
         table_sp, acc_sp, sem0, sem1) = refs
    cid = lax.axis_index("c")
    tid = lax.axis_index("s")

    def fill(i, _):
        for g in range(FH // 16):
            rows_v0[i, pl.ds(g * 16, 16)] = jnp.zeros((16,), jnp.float32)
        if compute_deg:
            ones_v[i, :] = jnp.ones((DEGW,), jnp.float32)
        return 0
    lax.fori_loop(0, CHUNK, fill, 0)

    my0 = tid * ROWS_PER_TILE
    for k in range(ROWS_PER_TILE // CHUNK):
        pltpu.sync_copy(rows_v0, acc_sp.at[pl.ds(my0 + k * CHUNK, CHUNK)])
        if compute_deg:
            pltpu.sync_copy(rows_v0.at[pl.ds(0, CHUNK), pl.ds(0, DEGW)],
                            deg_sp.at[pl.ds(my0 + k * CHUNK, CHUNK)])

    @pl.when(cid == 0)
    def _():
        pltpu.sync_copy(h_hbm.at[pl.ds(my0, ROWS_PER_TILE), pl.ds(0, FH)],
                        table_sp.at[pl.ds(my0, ROWS_PER_TILE)])

    @pl.when(cid == 1)
    def _():
        pltpu.sync_copy(h_hbm.at[pl.ds(my0, ROWS_PER_TILE), pl.ds(FH, FH)],
                        table_sp.at[pl.ds(my0, ROWS_PER_TILE)])
    plsc.subcore_barrier()

    def outer(g, _):
        base = tid * CHUNKS_PER_SUBCORE + g * IDXB
        pltpu.sync_copy(src_hbm.at[pl.ds(base, IDXB)], src_v)
        pltpu.sync_copy(dst_hbm.at[pl.ds(base, IDXB)], dst_v)
        pltpu.async_copy(table_sp.at[src_v.at[0]], rows_v0, sem0)

        def inner(jj, _):
            j = jj * 2
            pltpu.make_async_copy(table_sp.at[src_v.at[0]], rows_v0, sem0).wait()
            pltpu.async_copy(table_sp.at[src_v.at[j + 1]], rows_v1, sem1)
            pltpu.sync_copy(rows_v0, acc_sp.at[dst_v.at[j]], add=True)
            if compute_deg:
                @pl.when((g + jj) % 2 == cid)
                def _():
                    pltpu.sync_copy(ones_v, deg_sp.at[dst_v.at[j]], add=True)

            pltpu.make_async_copy(table_sp.at[src_v.at[0]], rows_v1, sem1).wait()

            @pl.when(jj + 1 < IDXB // 2)
            def _():
                pltpu.async_copy(table_sp.at[src_v.at[j + 2]], rows_v0, sem0)
            pltpu.sync_copy(rows_v1, acc_sp.at[dst_v.at[j + 1]], add=True)
            if compute_deg:
                @pl.when((g + jj) % 2 != cid)
                def _():
                    pltpu.sync_copy(ones_v, deg_sp.at[dst_v.at[j + 1]], add=True)
            return 0
        lax.fori_loop(0, IDXB // 2, inner, 0)
        return 0
    lax.fori_loop(0, CHUNKS_PER_SUBCORE // IDXB, outer, 0)

    plsc.subcore_barrier()

    out0 = cid * NPAD + my0
    pltpu.sync_copy(acc_sp.at[pl.ds(my0, ROWS_PER_TILE)],
                    agg_out.at[pl.ds(out0, ROWS_PER_TILE)])
    if compute_deg:
        pltpu.sync_copy(deg_sp.at[pl.ds(my0, ROWS_PER_TILE)],
                        deg_out.at[pl.ds(out0, ROWS_PER_TILE)])


_SC_PARAMS = pltpu.CompilerParams(use_tc_tiling_on_sc=False)
_SC_MESH = dict(core_axis_name="c", subcore_axis_name="s")

def _make_sc_agg(compute_deg):
    out_type = [jax.ShapeDtypeStruct((2 * NPAD, FH), jnp.float32)]
    scratch = [
        pltpu.VMEM((IDXB, CHUNK), jnp.int32),
        pltpu.VMEM((IDXB, CHUNK), jnp.int32),
        pltpu.VMEM((CHUNK, FH), jnp.float32),
        pltpu.VMEM((CHUNK, FH), jnp.float32),
    ]
    if compute_deg:
        out_type.append(jax.ShapeDtypeStruct((2 * NPAD, DEGW), jnp.float32))
        scratch.append(pltpu.VMEM((CHUNK, DEGW), jnp.float32))
    scratch.append(pltpu.VMEM_SHARED((NPAD, FH), jnp.float32))
    scratch.append(pltpu.VMEM_SHARED((NPAD, FH), jnp.float32))
    if compute_deg:
        scratch.append(pltpu.VMEM_SHARED((NPAD, DEGW), jnp.float32))
    scratch += [pltpu.SemaphoreType.DMA, pltpu.SemaphoreType.DMA]
    return pl.kernel(
        functools.partial(_sc_agg_body, compute_deg),
        out_type=out_type if compute_deg else out_type[0],
        mesh=plsc.VectorSubcoreMesh(**_SC_MESH),
        scratch_types=scratch,
        compiler_params=_SC_PARAMS,
    )


_sc_agg_deg = _make_sc_agg(True)
_sc_agg = _make_sc_agg(False)


def _sc_deg_body(dst_hbm, deg_out, dst_v, ones_v, zbuf_v, deg_sp):
    cid = lax.axis_index("c")
    tid = lax.axis_index("s")
    wid = cid * 16 + tid

    def fill(i, _):
        ones_v[i, :] = jnp.ones((DEGW,), jnp.float32)
        zbuf_v[i, :] = jnp.zeros((DEGW,), jnp.float32)
        return 0
    lax.fori_loop(0, CHUNK, fill, 0)

    my0 = tid * ROWS_PER_TILE
    for k in range(ROWS_PER_TILE // CHUNK):
        pltpu.sync_copy(zbuf_v, deg_sp.at[pl.ds(my0 + k * CHUNK, CHUNK)])
    plsc.subcore_barrier()

    pltpu.sync_copy(dst_hbm.at[pl.ds(wid * CHUNKS_PER_TILE, CHUNKS_PER_TILE)], dst_v)

    def edge_body(j, _):
        pltpu.sync_copy(ones_v, deg_sp.at[dst_v.at[j]], add=True)
        return 0
    lax.fori_loop(0, CHUNKS_PER_TILE, edge_body, 0)

    plsc.subcore_barrier()

    out0 = cid * NPAD + my0
    pltpu.sync_copy(deg_sp.at[pl.ds(my0, ROWS_PER_TILE)],
                    deg_out.at[pl.ds(out0, ROWS_PER_TILE)])


_sc_deg = pl.kernel(
    _sc_deg_body,
    out_type=jax.ShapeDtypeStruct((2 * NPAD, DEGW), jnp.float32),
    mesh=plsc.VectorSubcoreMesh(**_SC_MESH),
    scratch_types=[
        pltpu.VMEM((CHUNKS_PER_TILE, CHUNK), jnp.int32),
        pltpu.VMEM((CHUNK, DEGW), jnp.float32),
        pltpu.VMEM((CHUNK, DEGW), jnp.float32),
        pltpu.VMEM_SHARED((NPAD, DEGW), jnp.float32),
    ],
    compiler_params=_SC_PARAMS,
)

_BLK = 1280
_NB = NPAD // _BLK


def _tc_layer1_body(x, a0, a1, d0, d1, wl, wr0, wr1, b, o):
    inv = 1.0 / jnp.clip(d0[..., :1] + d1[..., :1], 1.0, None)
    h = (jnp.dot(x[...], wl[...], preferred_element_type=jnp.float32)
         + jnp.dot(a0[...] * inv, wr0[...], preferred_element_type=jnp.float32)
         + jnp.dot(a1[...] * inv, wr1[...], preferred_element_type=jnp.float32)
         + b[...])
    o[...] = jnp.maximum(h, 0.0)


def _tc_layer2_body(x, a0, a1, d0, d1, wl, wr0, wr1, b, wm, bm, o):
    inv = 1.0 / jnp.clip(d0[..., :1] + d1[..., :1], 1.0, None)
    h = (jnp.dot(x[...], wl[...], preferred_element_type=jnp.float32)
         + jnp.dot(a0[...] * inv, wr0[...], preferred_element_type=jnp.float32)
         + jnp.dot(a1[...] * inv, wr1[...], preferred_element_type=jnp.float32)
         + b[...])
    h = jnp.maximum(h, 0.0)
    logits = jnp.dot(h, wm[...], preferred_element_type=jnp.float32) + bm[...]
    m = jnp.max(logits, axis=1, keepdims=True)
    s = logits - m
    lse = jnp.log(jnp.sum(jnp.exp(s), axis=1, keepdims=True))
    o[...] = s - lse


def _row_spec(w):
    return pl.BlockSpec((_BLK, w), lambda i: (i, 0))


def _hi_spec(w):
    return pl.BlockSpec((_BLK, w), lambda i: (i + _NB, 0))


def _full_spec(h, w):
    return pl.BlockSpec((h, w), lambda i: (0, 0))


def _tc_layer1(x, agg, deg, wl, wr0, wr1, b):
    return pl.pallas_call(
        _tc_layer1_body,
        grid=(_NB,),
        in_specs=[_row_spec(F), _row_spec(FH), _hi_spec(FH),
                  _row_spec(DEGW), _hi_spec(DEGW),
                  _full_spec(F, F), _full_spec(FH, F), _full_spec(FH, F),
                  _full_spec(1, F)],
        out_specs=_row_spec(F),
        out_shape=jax.ShapeDtypeStruct((NPAD, F), jnp.float32),
    )(x, agg, agg, deg, deg, wl, wr0, wr1, b)


def _tc_layer2(x, agg, deg, wl, wr0, wr1, b, wm, bm):
    return pl.pallas_call(
        _tc_layer2_body,
        grid=(_NB,),
        in_specs=[_row_spec(F), _row_spec(FH), _hi_spec(FH),
                  _row_spec(DEGW), _hi_spec(DEGW),
                  _full_spec(F, F), _full_spec(FH, F), _full_spec(FH, F),
                  _full_spec(1, F), _full_spec(F, C), _full_spec(1, C)],
        out_specs=_row_spec(C),
        out_shape=jax.ShapeDtypeStruct((NPAD, C), jnp.float32),
    )(x, agg, agg, deg, deg, wl, wr0, wr1, b, wm, bm)


def kernel(x, edge_index, W1l, W1r, b1, W2l, W2r, b2, Wm, bm):
    src = edge_index[0]
    dst = edge_index[1]
    pad = EPAD - E
    src_p = jnp.concatenate([src, jnp.zeros((pad,), jnp.int32)]).reshape(NCHK, CHUNK)
    dst_p = jnp.concatenate([dst, jnp.full((pad,), N, jnp.int32)]).reshape(NCHK, CHUNK)

    xp = jnp.concatenate([x, jnp.zeros((NPAD - N, F), jnp.float32)])

    agg1 = _sc_agg(xp, src_p, dst_p)
    deg = _sc_deg(dst_p)

    h1 = _tc_layer1(xp, agg1, deg, W1l, W1r[:FH], W1r[FH:], b1.reshape(1, F))

    agg2 = _sc_agg(h1, src_p, dst_p)

    out = _tc_layer2(h1, agg2, deg, W2l, W2r[:FH], W2r[FH:],
                     b2.reshape(1, F), Wm, bm.reshape(1, C))
    return out[:N]


# speedup vs baseline: 1.0904x; 1.0483x over previous
import functools

import jax
import jax.numpy as jnp
from jax import lax
from jax.experimental import pallas as pl
from jax.experimental.pallas import tpu as pltpu
from jax.experimental.pallas import tpu_sc as plsc

N = 10000
E = 320000
F = 128
C = 64

NPAD = 10240
ROWS_PER_TILE = NPAD // 16
CHUNK = 128
NCHK = ((E + CHUNK - 1) // CHUNK + 255) // 256 * 256
EPAD = NCHK * CHUNK
CHUNKS_PER_TILE = NCHK // 32
CHUNKS_PER_SUBCORE = NCHK // 16
FH = F // 2
IDXB = 16
DEGW = 16


def _sc_agg_body(compute_deg, h_hbm, src_hbm, dst_hbm, *refs):
    if compute_deg:
        (agg_out, deg_out, src_v, dst_v, rows_v0, rows_v1, ones_v,
         table_sp, acc_sp, deg_sp, sem0, sem1) = refs
    else:
        (agg_out, src_v, dst_v, rows_v0, rows_v1,
         table_sp, acc_sp, sem0, sem1) = refs
    cid = lax.axis_index("c")
    tid = lax.axis_index("s")

    def fill(i, _):
        for g in range(FH // 16):
            rows_v0[i, pl.ds(g * 16, 16)] = jnp.zeros((16,), jnp.float32)
        if compute_deg:
            ones_v[i, :] = jnp.ones((DEGW,), jnp.float32)
        return 0
    lax.fori_loop(0, CHUNK, fill, 0)

    my0 = tid * ROWS_PER_TILE
    for k in range(ROWS_PER_TILE // CHUNK):
        pltpu.sync_copy(rows_v0, acc_sp.at[pl.ds(my0 + k * CHUNK, CHUNK)])
        if compute_deg:
            pltpu.sync_copy(rows_v0.at[pl.ds(0, CHUNK), pl.ds(0, DEGW)],
                            deg_sp.at[pl.ds(my0 + k * CHUNK, CHUNK)])

    @pl.when(cid == 0)
    def _():
        pltpu.sync_copy(h_hbm.at[pl.ds(my0, ROWS_PER_TILE), pl.ds(0, FH)],
                        table_sp.at[pl.ds(my0, ROWS_PER_TILE)])

    @pl.when(cid == 1)
    def _():
        pltpu.sync_copy(h_hbm.at[pl.ds(my0, ROWS_PER_TILE), pl.ds(FH, FH)],
                        table_sp.at[pl.ds(my0, ROWS_PER_TILE)])
    plsc.subcore_barrier()

    def outer(g, _):
        base = tid * CHUNKS_PER_SUBCORE + g * IDXB
        pltpu.sync_copy(src_hbm.at[pl.ds(base, IDXB)], src_v)
        pltpu.sync_copy(dst_hbm.at[pl.ds(base, IDXB)], dst_v)
        pltpu.async_copy(table_sp.at[src_v.at[0]], rows_v0, sem0)

        def inner(jj, _):
            j = jj * 2
            pltpu.make_async_copy(table_sp.at[src_v.at[0]], rows_v0, sem0).wait()
            pltpu.async_copy(table_sp.at[src_v.at[j + 1]], rows_v1, sem1)
            pltpu.sync_copy(rows_v0, acc_sp.at[dst_v.at[j]], add=True)
            if compute_deg:
                @pl.when((g + jj) % 2 == cid)
                def _():
                    pltpu.sync_copy(ones_v, deg_sp.at[dst_v.at[j]], add=True)

            pltpu.make_async_copy(table_sp.at[src_v.at[0]], rows_v1, sem1).wait()

            @pl.when(jj + 1 < IDXB // 2)
            def _():
                pltpu.async_copy(table_sp.at[src_v.at[j + 2]], rows_v0, sem0)
            pltpu.sync_copy(rows_v1, acc_sp.at[dst_v.at[j + 1]], add=True)
            if compute_deg:
                @pl.when((g + jj) % 2 != cid)
                def _():
                    pltpu.sync_copy(ones_v, deg_sp.at[dst_v.at[j + 1]], add=True)
            return 0
        lax.fori_loop(0, IDXB // 2, inner, 0)
        return 0
    lax.fori_loop(0, CHUNKS_PER_SUBCORE // IDXB, outer, 0)

    plsc.subcore_barrier()

    @pl.when(cid == 0)
    def _():
        pltpu.sync_copy(acc_sp.at[pl.ds(my0, ROWS_PER_TILE)],
                        agg_out.at[pl.ds(my0, ROWS_PER_TILE), pl.ds(0, FH)])

    @pl.when(cid == 1)
    def _():
        pltpu.sync_copy(acc_sp.at[pl.ds(my0, ROWS_PER_TILE)],
                        agg_out.at[pl.ds(my0, ROWS_PER_TILE), pl.ds(FH, FH)])
    if compute_deg:
        pltpu.sync_copy(deg_sp.at[pl.ds(my0, ROWS_PER_TILE)],
                        deg_out.at[pl.ds(cid * NPAD + my0, ROWS_PER_TILE)])


_SC_PARAMS = pltpu.CompilerParams(use_tc_tiling_on_sc=False)
_SC_MESH = dict(core_axis_name="c", subcore_axis_name="s")

def _make_sc_agg(compute_deg):
    out_type = [jax.ShapeDtypeStruct((NPAD, F), jnp.float32)]
    scratch = [
        pltpu.VMEM((IDXB, CHUNK), jnp.int32),
        pltpu.VMEM((IDXB, CHUNK), jnp.int32),
        pltpu.VMEM((CHUNK, FH), jnp.float32),
        pltpu.VMEM((CHUNK, FH), jnp.float32),
    ]
    if compute_deg:
        out_type.append(jax.ShapeDtypeStruct((2 * NPAD, DEGW), jnp.float32))
        scratch.append(pltpu.VMEM((CHUNK, DEGW), jnp.float32))
    scratch.append(pltpu.VMEM_SHARED((NPAD, FH), jnp.float32))
    scratch.append(pltpu.VMEM_SHARED((NPAD, FH), jnp.float32))
    if compute_deg:
        scratch.append(pltpu.VMEM_SHARED((NPAD, DEGW), jnp.float32))
    scratch += [pltpu.SemaphoreType.DMA, pltpu.SemaphoreType.DMA]
    return pl.kernel(
        functools.partial(_sc_agg_body, compute_deg),
        out_type=out_type if compute_deg else out_type[0],
        mesh=plsc.VectorSubcoreMesh(**_SC_MESH),
        scratch_types=scratch,
        compiler_params=_SC_PARAMS,
    )


_sc_agg_deg = _make_sc_agg(True)
_sc_agg = _make_sc_agg(False)


def _sc_deg_body(dst_hbm, deg_out, dst_v, ones_v, zbuf_v, deg_sp):
    cid = lax.axis_index("c")
    tid = lax.axis_index("s")
    wid = cid * 16 + tid

    def fill(i, _):
        ones_v[i, :] = jnp.ones((DEGW,), jnp.float32)
        zbuf_v[i, :] = jnp.zeros((DEGW,), jnp.float32)
        return 0
    lax.fori_loop(0, CHUNK, fill, 0)

    my0 = tid * ROWS_PER_TILE
    for k in range(ROWS_PER_TILE // CHUNK):
        pltpu.sync_copy(zbuf_v, deg_sp.at[pl.ds(my0 + k * CHUNK, CHUNK)])
    plsc.subcore_barrier()

    pltpu.sync_copy(dst_hbm.at[pl.ds(wid * CHUNKS_PER_TILE, CHUNKS_PER_TILE)], dst_v)

    def edge_body(j, _):
        pltpu.sync_copy(ones_v, deg_sp.at[dst_v.at[j]], add=True)
        return 0
    lax.fori_loop(0, CHUNKS_PER_TILE, edge_body, 0)

    plsc.subcore_barrier()

    out0 = cid * NPAD + my0
    pltpu.sync_copy(deg_sp.at[pl.ds(my0, ROWS_PER_TILE), pl.ds(0, 8)],
                    deg_out.at[pl.ds(out0, ROWS_PER_TILE)])


_sc_deg = pl.kernel(
    _sc_deg_body,
    out_type=jax.ShapeDtypeStruct((2 * NPAD, 8), jnp.float32),
    mesh=plsc.VectorSubcoreMesh(**_SC_MESH),
    scratch_types=[
        pltpu.VMEM((CHUNKS_PER_TILE, CHUNK), jnp.int32),
        pltpu.VMEM((CHUNK, DEGW), jnp.float32),
        pltpu.VMEM((CHUNK, DEGW), jnp.float32),
        pltpu.VMEM_SHARED((NPAD, DEGW), jnp.float32),
    ],
    compiler_params=_SC_PARAMS,
)

_BLK = 1280
_NB = NPAD // _BLK


def _tc_layer1_body(x, a, d0, d1, wl, wr, b, o):
    inv = 1.0 / jnp.clip(d0[..., :1] + d1[..., :1], 1.0, None)
    h = (jnp.dot(x[...], wl[...], preferred_element_type=jnp.float32)
         + jnp.dot(a[...] * inv, wr[...], preferred_element_type=jnp.float32)
         + b[...])
    o[...] = jnp.maximum(h, 0.0)


def _tc_layer2_body(x, a, d0, d1, wl, wr, b, wm, bm, o):
    inv = 1.0 / jnp.clip(d0[..., :1] + d1[..., :1], 1.0, None)
    h = (jnp.dot(x[...], wl[...], preferred_element_type=jnp.float32)
         + jnp.dot(a[...] * inv, wr[...], preferred_element_type=jnp.float32)
         + b[...])
    h = jnp.maximum(h, 0.0)
    logits = jnp.dot(h, wm[...], preferred_element_type=jnp.float32) + bm[...]
    m = jnp.max(logits, axis=1, keepdims=True)
    s = logits - m
    lse = jnp.log(jnp.sum(jnp.exp(s), axis=1, keepdims=True))
    o[...] = s - lse


def _row_spec(w):
    return pl.BlockSpec((_BLK, w), lambda i: (i, 0))


def _hi_spec(w):
    return pl.BlockSpec((_BLK, w), lambda i: (i + _NB, 0))


def _full_spec(h, w):
    return pl.BlockSpec((h, w), lambda i: (0, 0))


def _tc_layer1(x, a, deg, wl, wr, b):
    return pl.pallas_call(
        _tc_layer1_body,
        grid=(_NB,),
        in_specs=[_row_spec(F), _row_spec(F),
                  _row_spec(8), _hi_spec(8),
                  _full_spec(F, F), _full_spec(F, F), _full_spec(1, F)],
        out_specs=_row_spec(F),
        out_shape=jax.ShapeDtypeStruct((NPAD, F), jnp.float32),
    )(x, a, deg, deg, wl, wr, b)


def _tc_layer2(x, a, deg, wl, wr, b, wm, bm):
    return pl.pallas_call(
        _tc_layer2_body,
        grid=(_NB,),
        in_specs=[_row_spec(F), _row_spec(F),
                  _row_spec(8), _hi_spec(8),
                  _full_spec(F, F), _full_spec(F, F), _full_spec(1, F),
                  _full_spec(F, C), _full_spec(1, C)],
        out_specs=_row_spec(C),
        out_shape=jax.ShapeDtypeStruct((NPAD, C), jnp.float32),
    )(x, a, deg, deg, wl, wr, b, wm, bm)


def kernel(x, edge_index, W1l, W1r, b1, W2l, W2r, b2, Wm, bm):
    src = edge_index[0]
    dst = edge_index[1]
    pad = EPAD - E
    src_p = jnp.concatenate([src, jnp.zeros((pad,), jnp.int32)]).reshape(NCHK, CHUNK)
    dst_p = jnp.concatenate([dst, jnp.full((pad,), N, jnp.int32)]).reshape(NCHK, CHUNK)

    xp = jnp.concatenate([x, jnp.zeros((NPAD - N, F), jnp.float32)])

    agg1 = _sc_agg(xp, src_p, dst_p)
    deg = _sc_deg(dst_p)

    h1 = _tc_layer1(xp, agg1, deg, W1l, W1r, b1.reshape(1, F))

    agg2 = _sc_agg(h1, src_p, dst_p)

    out = _tc_layer2(h1, agg2, deg, W2l, W2r, b2.reshape(1, F),
                     Wm, bm.reshape(1, C))
    return out[:N]
